# baseline (speedup 1.0000x reference)
import functools

import jax
import jax.numpy as jnp
from jax import lax
from jax.experimental import pallas as pl
from jax.experimental.pallas import tpu as pltpu
from jax.experimental.pallas import tpu_sc as plsc

_N = 10000
_NP = 10240
_E = 320000
_E2 = _E + _N
_T = 12
_H = 128
_NW = 32
_K = 128
_CH = 82
_K2 = 64
_CH2 = 2 * _CH
_EPW = _CH * _K
_E2P = _NW * _EPW
_RPW = _NP // 16

_mesh = plsc.VectorSubcoreMesh(core_axis_name="c", subcore_axis_name="s")


def _softmax_body(ew_ref, out_ref):
    v = ew_ref[...]
    m = jnp.max(v)
    e = jnp.exp(v - m)
    out_ref[...] = e / jnp.sum(e)


def _softmax(ew2d):
    return pl.pallas_call(
        _softmax_body,
        out_shape=jax.ShapeDtypeStruct(ew2d.shape, jnp.float32),
    )(ew2d)


def _dinv_body(dp_ref, out_ref):
    d = jnp.sum(dp_ref[...], axis=0)
    out_ref[...] = jnp.where(d > 0, lax.rsqrt(d), 0.0).reshape(_NP // 128, 128)


def _dinv(deg_part):
    return pl.pallas_call(
        _dinv_body,
        out_shape=jax.ShapeDtypeStruct((_NP // 128, 128), jnp.float32),
    )(deg_part)


def _dense_core(xb, w0, w1, w2, tb, g, b, gw):
    tt, nb, _ = xb.shape
    y = xb.reshape(tt * nb, -1)
    a = jnp.dot(y, w0, preferred_element_type=jnp.float32).reshape(tt, nb, -1)
    bb = jnp.dot(y, w1, preferred_element_type=jnp.float32).reshape(tt, nb, -1)
    cc = jnp.dot(y, w2, preferred_element_type=jnp.float32).reshape(tt, nb, -1)
    z = jnp.zeros((1, nb, a.shape[-1]), jnp.float32)
    xc = bb + jnp.concatenate([z, a[:-1]], 0) + jnp.concatenate([cc[1:], z], 0)
    xc = jnp.clip(xc + tb.reshape(1, 1, -1), -10.0, 10.0)
    m = jnp.mean(xc, axis=0, keepdims=True)
    v = jnp.mean((xc - m) ** 2, axis=0, keepdims=True)
    xc = (xc - m) * lax.rsqrt(v + 1e-5) * g.reshape(1, 1, -1) + b.reshape(1, 1, -1)
    xc = jnp.maximum(xc, 0.0)
    xw = jnp.dot(xc.reshape(tt * nb, -1), gw, preferred_element_type=jnp.float32)
    return xw.reshape(tt, nb, -1)


def _dense0_body(x_ref, w0_ref, w1_ref, w2_ref, tb_ref, g_ref, b_ref, gw_ref, out_ref):
    out_ref[...] = _dense_core(x_ref[...], w0_ref[...], w1_ref[...], w2_ref[...],
                               tb_ref[...], g_ref[...], b_ref[...], gw_ref[...])


def _merge_agg(agg, gb):
    h = agg[0] + agg[1] + gb.reshape(1, 1, -1)
    return jnp.clip(jnp.maximum(h, 0.0), -10.0, 10.0)


def _dense1_body(agg_ref, gb_ref, w0_ref, w1_ref, w2_ref, tb_ref, g_ref, b_ref,
                 gw_ref, out_ref):
    h = _merge_agg(agg_ref[...], gb_ref[...])
    out_ref[...] = _dense_core(h, w0_ref[...], w1_ref[...], w2_ref[...],
                               tb_ref[...], g_ref[...], b_ref[...], gw_ref[...])


def _out_body(agg_ref, gb_ref, ow_ref, ob_ref, out_ref):
    h = _merge_agg(agg_ref[...], gb_ref[...])
    hbar = jnp.mean(h, axis=0)
    out_ref[...] = jnp.dot(hbar, ow_ref[...], preferred_element_type=jnp.float32) \
        + ob_ref[...].reshape(1, -1)


_NB = 512
_GRID = _NP // _NB


def _wspec(shape):
    nd = len(shape)
    return pl.BlockSpec(shape, lambda i: (0,) * nd)


def _dense0(xpad, w0, w1, w2, tb, g, b, gw):
    return pl.pallas_call(
        _dense0_body,
        grid=(_GRID,),
        in_specs=[
            pl.BlockSpec((_T, _NB, _H), lambda i: (0, i, 0)),
            _wspec((_H, _H)), _wspec((_H, _H)), _wspec((_H, _H)),
            _wspec((1, _H)), _wspec((1, _H)), _wspec((1, _H)), _wspec((_H, _H)),
        ],
        out_specs=pl.BlockSpec((_T, _NB, _H), lambda i: (0, i, 0)),
        out_shape=jax.ShapeDtypeStruct((_T, _NP, _H), jnp.float32),
    )(xpad, w0, w1, w2, tb, g, b, gw)


def _dense1(agg, gb, w0, w1, w2, tb, g, b, gw):
    return pl.pallas_call(
        _dense1_body,
        grid=(_GRID,),
        in_specs=[
            pl.BlockSpec((2, _T, _NB, _H), lambda i: (0, 0, i, 0)),
            _wspec((1, _H)),
            _wspec((_H, _H)), _wspec((_H, _H)), _wspec((_H, _H)),
            _wspec((1, _H)), _wspec((1, _H)), _wspec((1, _H)), _wspec((_H, _H)),
        ],
        out_specs=pl.BlockSpec((_T, _NB, _H), lambda i: (0, i, 0)),
        out_shape=jax.ShapeDtypeStruct((_T, _NP, _H), jnp.float32),
    )(agg, gb, w0, w1, w2, tb, g, b, gw)


def _proj_out(agg, gb, ow, ob):
    return pl.pallas_call(
        _out_body,
        grid=(_GRID,),
        in_specs=[
            pl.BlockSpec((2, _T, _NB, _H), lambda i: (0, 0, i, 0)),
            _wspec((1, _H)), _wspec((_H, _H)), _wspec((1, _H)),
        ],
        out_specs=pl.BlockSpec((_NB, _H), lambda i: (i, 0)),
        out_shape=jax.ShapeDtypeStruct((_NP, _H), jnp.float32),
    )(agg, gb, ow, ob)


@functools.partial(
    pl.kernel,
    out_type=jax.ShapeDtypeStruct((_NW, _NP), jnp.float32),
    mesh=_mesh,
    compiler_params=pltpu.CompilerParams(needs_layout_passes=False),
    scratch_types=[
        pltpu.VMEM((_EPW,), jnp.int32),
        pltpu.VMEM((_EPW,), jnp.float32),
        pltpu.VMEM((_NP,), jnp.float32),
    ],
)
def _deg_kernel(dst_hbm, ew_hbm, out_hbm, dstv, ewv, degv):
    c = lax.axis_index("c")
    s = lax.axis_index("s")
    w = c * 16 + s
    pltpu.sync_copy(dst_hbm.at[w], dstv)
    pltpu.sync_copy(ew_hbm.at[w], ewv)

    def zero(i, carry):
        degv[pl.ds(i * 16, 16)] = jnp.zeros((16,), jnp.float32)
        return carry
    lax.fori_loop(0, _NP // 16, zero, 0)

    def step(i, carry):
        idx = dstv[pl.ds(i * 16, 16)]
        wv = ewv[pl.ds(i * 16, 16)]
        plsc.addupdate_scatter(degv, [idx], wv)
        return carry
    lax.fori_loop(0, _EPW // 16, step, 0)
    pltpu.sync_copy(degv, out_hbm.at[w])


@functools.partial(
    pl.kernel,
    out_type=jax.ShapeDtypeStruct((_NW, _EPW), jnp.float32),
    mesh=_mesh,
    compiler_params=pltpu.CompilerParams(needs_layout_passes=False),
    scratch_types=[
        pltpu.VMEM((_EPW,), jnp.int32),
        pltpu.VMEM((_EPW,), jnp.int32),
        pltpu.VMEM((_EPW,), jnp.float32),
        pltpu.VMEM((_NP,), jnp.float32),
        pltpu.VMEM((_EPW,), jnp.float32),
    ],
)
def _norm_kernel(src_hbm, dst_hbm, ew_hbm, dinv_hbm, out_hbm,
                 srcv, dstv, ewv, dinvv, outv):
    c = lax.axis_index("c")
    s = lax.axis_index("s")
    w = c * 16 + s
    pltpu.sync_copy(src_hbm.at[w], srcv)
    pltpu.sync_copy(dst_hbm.at[w], dstv)
    pltpu.sync_copy(ew_hbm.at[w], ewv)
    pltpu.sync_copy(dinv_hbm, dinvv)

    def step(i, carry):
        sl = pl.ds(i * 16, 16)
        a = plsc.load_gather(dinvv, [srcv[sl]])
        bq = plsc.load_gather(dinvv, [dstv[sl]])
        outv[sl] = a * ewv[sl] * bq
        return carry
    lax.fori_loop(0, _EPW // 16, step, 0)
    pltpu.sync_copy(outv, out_hbm.at[w])


@functools.partial(
    pl.kernel,
    out_type=jax.ShapeDtypeStruct((2, _T, _NP, _H), jnp.float32),
    mesh=_mesh,
    compiler_params=pltpu.CompilerParams(needs_layout_passes=False),
    scratch_types=[
        pltpu.VMEM((_CH, _K), jnp.int32),
        pltpu.VMEM((_K2,), jnp.int32),
        pltpu.VMEM((_K2,), jnp.int32),
        pltpu.VMEM((_K2,), jnp.int32),
        pltpu.VMEM((_K2,), jnp.int32),
        pltpu.VMEM((_K2,), jnp.float32),
        pltpu.VMEM((_K2,), jnp.float32),
        pltpu.VMEM((_K2,), jnp.float32),
        pltpu.VMEM((_K2,), jnp.float32),
        pltpu.VMEM((_K2, _H), jnp.float32),
        pltpu.VMEM((_K2, _H), jnp.float32),
        pltpu.VMEM((_K2, _H), jnp.float32),
        pltpu.VMEM((_K2, _H), jnp.float32),
        pltpu.VMEM((16, _H), jnp.float32),
        pltpu.VMEM_SHARED((_NP, _H), jnp.float32),
        [pltpu.SemaphoreType.DMA] * 4,
        [pltpu.SemaphoreType.DMA] * 4,
        [pltpu.SemaphoreType.DMA] * 4,
        [pltpu.SemaphoreType.DMA] * 4,
        pltpu.SemaphoreType.DMA,
    ],
)
def _mp_kernel(xw_hbm, src_hbm, dst_hbm, norm_hbm, out_hbm,
               idx2d, dstc0, dstc1, dstc2, dstc3,
               normc0, normc1, normc2, normc3,
               rows0, rows1, rows2, rows3, zbuf, acc,
               gsem, ssem, dsem, nsem, csem):
    c = lax.axis_index("c")
    s = lax.axis_index("s")
    w = c * 16 + s
    pltpu.sync_copy(src_hbm.at[w], idx2d)
    dstcs = (dstc0, dstc1, dstc2, dstc3)
    normcs = (normc0, normc1, normc2, normc3)
    rowss = (rows0, rows1, rows2, rows3)

    def zb(i, carry):
        zbuf[i // 8, pl.ds((i % 8) * 16, 16)] = jnp.zeros((16,), jnp.float32)
        return carry
    lax.fori_loop(0, 16 * 8, zb, 0)
    my0 = s * _RPW

    def gath(j, rows, gsem):
        idx = idx2d.at[j // 2, pl.ds((j % 2) * _K2, _K2)]
        return pltpu.make_async_copy(xw_hbm.at[idx], rows, gsem)

    def dfetch(j, r):
        return pltpu.make_async_copy(dst_hbm.at[w, j], dstcs[r], dsem[r])

    def nfetch(j, r):
        return pltpu.make_async_copy(norm_hbm.at[w, j], normcs[r], nsem[r])

    def scat(r):
        return pltpu.make_async_copy(rowss[r], acc.at[dstcs[r]], ssem[r])

    def prefetch(j, r):
        gath(j, rowss[r], gsem[r]).start()
        dfetch(j, r).start()
        nfetch(j, r).start()

    def scale(normc, rows):
        def body(q, cz):
            nv = normc[pl.ds(q * 16, 16)]
            for e in range(16):
                sc_ = nv[e]
                for k in range(_H // 16):
                    sl = pl.ds(k * 16, 16)
                    rows[q * 16 + e, sl] = rows[q * 16 + e, sl] * sc_
            return cz
        lax.fori_loop(0, _K2 // 16, body, 0)

    def clrcp(i):
        return pltpu.make_async_copy(zbuf, acc.at[pl.ds(my0 + i * 16, 16)], csem)

    def per_t(t, carry):
        def clr(i, cy):
            clrcp(i).start()
            return cy
        lax.fori_loop(0, _RPW // 16, clr, 0)

        def clw(i, cy):
            clrcp(i).wait()
            return cy
        lax.fori_loop(0, _RPW // 16, clw, 0)
        plsc.subcore_barrier()

        prefetch(0, 0)
        prefetch(1, 1)
        prefetch(2, 2)
        _NQ = _CH2 // 4

        def quad(i, cy):
            a = 4 * i
            for r in range(4):
                cch = a + r
                gath(cch, rowss[r], gsem[r]).wait()
                nfetch(cch, r).wait()
                scale(normcs[r], rowss[r])
                dfetch(cch, r).wait()
                pltpu.async_copy(rowss[r], acc.at[dstcs[r]], ssem[r], add=True)
                rp = (r + 3) % 4
                if r == 0:
                    @pl.when(i > 0)
                    def _():
                        scat(rp).wait()
                    prefetch(a + 3, rp)
                else:
                    @pl.when(i + 1 < _NQ)
                    def _():
                        scat(rp).wait()
                        prefetch(cch + 3, rp)
            return cy
        lax.fori_loop(0, _NQ, quad, 0)
        for r in range(4):
            scat(r).wait()
        plsc.subcore_barrier()
        pltpu.sync_copy(acc.at[pl.ds(my0, _RPW)],
                        out_hbm.at[c, t, pl.ds(my0, _RPW)])

        def bump(i, cy):
            r = i // 8
            sl = pl.ds((i % 8) * 16, 16)
            idx2d[r, sl] = idx2d[r, sl] + _NP
            return cy
        lax.fori_loop(0, _CH * (_K // 16), bump, 0)
        return carry
    lax.fori_loop(0, _T, per_t, 0)


def kernel(x, edge_index, edge_attr, l0_tw, l0_tb, l0_g, l0_b, l0_gw, l0_gb,
           l1_tw, l1_tb, l1_g, l1_b, l1_gw, l1_gb, out_w, out_b):
    f32 = jnp.float32
    ew = jnp.squeeze(edge_attr, axis=-1)
    ew_soft = _softmax(ew.reshape(_E // 128, 128)).reshape(_E)
    src = edge_index[0]
    dst = edge_index[1]
    loopi = jnp.arange(_N, dtype=jnp.int32)
    npad = _E2P - _E2
    src2 = jnp.concatenate([src, loopi, jnp.zeros((npad,), jnp.int32)])
    dst2 = jnp.concatenate([dst, loopi, jnp.zeros((npad,), jnp.int32)])
    ew2 = jnp.concatenate([ew_soft, jnp.ones((_N,), f32), jnp.zeros((npad,), f32)])
    srcp = src2.reshape(_NW, _EPW)
    dstp = dst2.reshape(_NW, _EPW)
    srcp3 = src2.reshape(_NW, _CH, _K)
    dstp3 = dst2.reshape(_NW, _CH2, _K2)
    ewp = ew2.reshape(_NW, _EPW)

    deg_part = _deg_kernel(dstp, ewp)
    dinv = _dinv(deg_part).reshape(_NP)
    normp = _norm_kernel(srcp, dstp, ewp, dinv)
    normp3 = normp.reshape(_NW, _CH2, _K2)

    xpad = jnp.pad(x, ((0, 0), (0, _NP - _N), (0, 0)))
    w00 = l0_tw[:, :, 0].T
    w01 = l0_tw[:, :, 1].T
    w02 = l0_tw[:, :, 2].T
    w10 = l1_tw[:, :, 0].T
    w11 = l1_tw[:, :, 1].T
    w12 = l1_tw[:, :, 2].T
    r1 = lambda a: a.reshape(1, _H)

    xw0 = _dense0(xpad, w00, w01, w02, r1(l0_tb), r1(l0_g), r1(l0_b), l0_gw)
    agg0 = _mp_kernel(xw0.reshape(_T * _NP, _H), srcp3, dstp3, normp3)
    xw1 = _dense1(agg0, r1(l0_gb), w10, w11, w12, r1(l1_tb), r1(l1_g), r1(l1_b), l1_gw)
    agg1 = _mp_kernel(xw1.reshape(_T * _NP, _H), srcp3, dstp3, normp3)
    outp = _proj_out(agg1, r1(l1_gb), out_w, r1(out_b))
    return outp[:_N]


# speedup vs baseline: 1.8766x; 1.8766x over previous
import functools

import jax
import jax.numpy as jnp
from jax import lax
from jax.experimental import pallas as pl
from jax.experimental.pallas import tpu as pltpu
from jax.experimental.pallas import tpu_sc as plsc

_N = 10000
_NP = 10240
_E = 320000
_E2 = _E + _N
_T = 12
_H = 128
_NW = 32
_K = 128
_CH = -(-_E2 // (_NW * _K))
_K2 = 64
_CH2 = 2 * _CH
_EPW = _CH * _K
_E2P = _NW * _EPW
_RPW = _NP // 16

_mesh = plsc.VectorSubcoreMesh(core_axis_name="c", subcore_axis_name="s")


def _softmax_body(ew_ref, out_ref):
    v = ew_ref[...]
    m = jnp.max(v)
    e = jnp.exp(v - m)
    out_ref[...] = e / jnp.sum(e)


def _softmax(ew2d):
    return pl.pallas_call(
        _softmax_body,
        out_shape=jax.ShapeDtypeStruct(ew2d.shape, jnp.float32),
    )(ew2d)


def _dinv_body(dp_ref, out_ref):
    d = jnp.sum(dp_ref[...], axis=0)
    out_ref[...] = jnp.where(d > 0, lax.rsqrt(d), 0.0).reshape(_NP // 128, 128)


def _dinv(deg_part):
    return pl.pallas_call(
        _dinv_body,
        out_shape=jax.ShapeDtypeStruct((_NP // 128, 128), jnp.float32),
    )(deg_part)


def _dense_core(xb, w0, w1, w2, tb, g, b, gw):
    tt, nb, _ = xb.shape
    y = xb.reshape(tt * nb, -1)
    a = jnp.dot(y, w0, preferred_element_type=jnp.float32).reshape(tt, nb, -1)
    bb = jnp.dot(y, w1, preferred_element_type=jnp.float32).reshape(tt, nb, -1)
    cc = jnp.dot(y, w2, preferred_element_type=jnp.float32).reshape(tt, nb, -1)
    z = jnp.zeros((1, nb, a.shape[-1]), jnp.float32)
    xc = bb + jnp.concatenate([z, a[:-1]], 0) + jnp.concatenate([cc[1:], z], 0)
    xc = jnp.clip(xc + tb.reshape(1, 1, -1), -10.0, 10.0)
    m = jnp.mean(xc, axis=0, keepdims=True)
    v = jnp.mean((xc - m) ** 2, axis=0, keepdims=True)
    xc = (xc - m) * lax.rsqrt(v + 1e-5) * g.reshape(1, 1, -1) + b.reshape(1, 1, -1)
    xc = jnp.maximum(xc, 0.0)
    xw = jnp.dot(xc.reshape(tt * nb, -1), gw, preferred_element_type=jnp.float32)
    return xw.reshape(tt, nb, -1)


def _dense0_body(x_ref, w0_ref, w1_ref, w2_ref, tb_ref, g_ref, b_ref, gw_ref, out_ref):
    out_ref[...] = _dense_core(x_ref[...], w0_ref[...], w1_ref[...], w2_ref[...],
                               tb_ref[...], g_ref[...], b_ref[...], gw_ref[...])


def _merge_agg(agg, gb):
    h = agg[0] + agg[1] + gb.reshape(1, 1, -1)
    return jnp.clip(jnp.maximum(h, 0.0), -10.0, 10.0)


def _dense1_body(agg_ref, gb_ref, w0_ref, w1_ref, w2_ref, tb_ref, g_ref, b_ref,
                 gw_ref, out_ref):
    h = _merge_agg(agg_ref[...], gb_ref[...])
    out_ref[...] = _dense_core(h, w0_ref[...], w1_ref[...], w2_ref[...],
                               tb_ref[...], g_ref[...], b_ref[...], gw_ref[...])


def _out_body(agg_ref, gb_ref, ow_ref, ob_ref, out_ref):
    h = _merge_agg(agg_ref[...], gb_ref[...])
    hbar = jnp.mean(h, axis=0)
    out_ref[...] = jnp.dot(hbar, ow_ref[...], preferred_element_type=jnp.float32) \
        + ob_ref[...].reshape(1, -1)


_NB = 512
_GRID = _NP // _NB


def _wspec(shape):
    nd = len(shape)
    return pl.BlockSpec(shape, lambda i: (0,) * nd)


def _dense0(xpad, w0, w1, w2, tb, g, b, gw):
    return pl.pallas_call(
        _dense0_body,
        grid=(_GRID,),
        in_specs=[
            pl.BlockSpec((_T, _NB, _H), lambda i: (0, i, 0)),
            _wspec((_H, _H)), _wspec((_H, _H)), _wspec((_H, _H)),
            _wspec((1, _H)), _wspec((1, _H)), _wspec((1, _H)), _wspec((_H, _H)),
        ],
        out_specs=pl.BlockSpec((_T, _NB, _H), lambda i: (0, i, 0)),
        out_shape=jax.ShapeDtypeStruct((_T, _NP, _H), jnp.float32),
    )(xpad, w0, w1, w2, tb, g, b, gw)


def _dense1(agg, gb, w0, w1, w2, tb, g, b, gw):
    return pl.pallas_call(
        _dense1_body,
        grid=(_GRID,),
        in_specs=[
            pl.BlockSpec((2, _T, _NB, _H), lambda i: (0, 0, i, 0)),
            _wspec((1, _H)),
            _wspec((_H, _H)), _wspec((_H, _H)), _wspec((_H, _H)),
            _wspec((1, _H)), _wspec((1, _H)), _wspec((1, _H)), _wspec((_H, _H)),
        ],
        out_specs=pl.BlockSpec((_T, _NB, _H), lambda i: (0, i, 0)),
        out_shape=jax.ShapeDtypeStruct((_T, _NP, _H), jnp.float32),
    )(agg, gb, w0, w1, w2, tb, g, b, gw)


def _proj_out(agg, gb, ow, ob):
    return pl.pallas_call(
        _out_body,
        grid=(_GRID,),
        in_specs=[
            pl.BlockSpec((2, _T, _NB, _H), lambda i: (0, 0, i, 0)),
            _wspec((1, _H)), _wspec((_H, _H)), _wspec((1, _H)),
        ],
        out_specs=pl.BlockSpec((_NB, _H), lambda i: (i, 0)),
        out_shape=jax.ShapeDtypeStruct((_NP, _H), jnp.float32),
    )(agg, gb, ow, ob)


@functools.partial(
    pl.kernel,
    out_type=jax.ShapeDtypeStruct((_NW, _NP), jnp.float32),
    mesh=_mesh,
    compiler_params=pltpu.CompilerParams(needs_layout_passes=False),
    scratch_types=[
        pltpu.VMEM((_EPW,), jnp.int32),
        pltpu.VMEM((_EPW,), jnp.float32),
        pltpu.VMEM((_NP,), jnp.float32),
    ],
)
def _deg_kernel(dst_hbm, ew_hbm, out_hbm, dstv, ewv, degv):
    c = lax.axis_index("c")
    s = lax.axis_index("s")
    w = c * 16 + s
    pltpu.sync_copy(dst_hbm.at[w], dstv)
    pltpu.sync_copy(ew_hbm.at[w], ewv)

    def zero(i, carry):
        degv[pl.ds(i * 16, 16)] = jnp.zeros((16,), jnp.float32)
        return carry
    lax.fori_loop(0, _NP // 16, zero, 0)

    def step(i, carry):
        idx = dstv[pl.ds(i * 16, 16)]
        wv = ewv[pl.ds(i * 16, 16)]
        plsc.addupdate_scatter(degv, [idx], wv)
        return carry
    lax.fori_loop(0, _EPW // 16, step, 0)
    pltpu.sync_copy(degv, out_hbm.at[w])


@functools.partial(
    pl.kernel,
    out_type=jax.ShapeDtypeStruct((_NW, _EPW), jnp.float32),
    mesh=_mesh,
    compiler_params=pltpu.CompilerParams(needs_layout_passes=False),
    scratch_types=[
        pltpu.VMEM((_EPW,), jnp.int32),
        pltpu.VMEM((_EPW,), jnp.int32),
        pltpu.VMEM((_EPW,), jnp.float32),
        pltpu.VMEM((_NP,), jnp.float32),
        pltpu.VMEM((_EPW,), jnp.float32),
    ],
)
def _norm_kernel(src_hbm, dst_hbm, ew_hbm, dinv_hbm, out_hbm,
                 srcv, dstv, ewv, dinvv, outv):
    c = lax.axis_index("c")
    s = lax.axis_index("s")
    w = c * 16 + s
    pltpu.sync_copy(src_hbm.at[w], srcv)
    pltpu.sync_copy(dst_hbm.at[w], dstv)
    pltpu.sync_copy(ew_hbm.at[w], ewv)
    pltpu.sync_copy(dinv_hbm, dinvv)

    def step(i, carry):
        sl = pl.ds(i * 16, 16)
        a = plsc.load_gather(dinvv, [srcv[sl]])
        bq = plsc.load_gather(dinvv, [dstv[sl]])
        outv[sl] = a * ewv[sl] * bq
        return carry
    lax.fori_loop(0, _EPW // 16, step, 0)
    pltpu.sync_copy(outv, out_hbm.at[w])


@functools.partial(
    pl.kernel,
    out_type=jax.ShapeDtypeStruct((2, _T, _NP, _H), jnp.float32),
    mesh=_mesh,
    compiler_params=pltpu.CompilerParams(needs_layout_passes=False),
    scratch_types=[
        pltpu.VMEM((_CH, _K), jnp.int32),
        pltpu.VMEM((_EPW,), jnp.float32),
        pltpu.VMEM((_K2,), jnp.int32),
        pltpu.VMEM((_K2,), jnp.int32),
        pltpu.VMEM((_K2,), jnp.int32),
        pltpu.VMEM((_K2, _H), jnp.float32),
        pltpu.VMEM((_K2, _H), jnp.float32),
        pltpu.VMEM((_K2, _H), jnp.float32),
        pltpu.VMEM((16, _H), jnp.float32),
        pltpu.VMEM_SHARED((_NP, _H), jnp.float32),
        pltpu.SemaphoreType.DMA,
        pltpu.SemaphoreType.DMA,
        pltpu.SemaphoreType.DMA,
        pltpu.SemaphoreType.DMA,
        pltpu.SemaphoreType.DMA,
        pltpu.SemaphoreType.DMA,
        pltpu.SemaphoreType.DMA,
        pltpu.SemaphoreType.DMA,
        pltpu.SemaphoreType.DMA,
        pltpu.SemaphoreType.DMA,
    ],
)
def _mp_kernel(xw_hbm, src_hbm, dst_hbm, norm_hbm, out_hbm,
               idx2d, normv, dstc0, dstc1, dstc2, rows0, rows1, rows2,
               zbuf, acc, gsem0, gsem1, gsem2, ssem0, ssem1, ssem2,
               dsem0, dsem1, dsem2, csem):
    c = lax.axis_index("c")
    s = lax.axis_index("s")
    w = c * 16 + s
    pltpu.sync_copy(src_hbm.at[w], idx2d)
    pltpu.sync_copy(norm_hbm.at[w], normv)

    def zb(i, carry):
        zbuf[i // 8, pl.ds((i % 8) * 16, 16)] = jnp.zeros((16,), jnp.float32)
        return carry
    lax.fori_loop(0, 16 * 8, zb, 0)
    my0 = s * _RPW

    def gath(j, rows, gsem):
        idx = idx2d.at[j // 2, pl.ds((j % 2) * _K2, _K2)]
        return pltpu.make_async_copy(xw_hbm.at[idx], rows, gsem)

    def dfetch(j, dstc, dsem):
        return pltpu.make_async_copy(dst_hbm.at[w, j], dstc, dsem)

    def scat(rows, dstc, ssem):
        return pltpu.make_async_copy(rows, acc.at[dstc], ssem)

    def scale(cidx, rows):
        def body(q, cz):
            nv = normv[pl.ds(cidx * _K2 + q * 16, 16)]
            for e in range(16):
                sc_ = nv[e]
                for k in range(_H // 16):
                    sl = pl.ds(k * 16, 16)
                    rows[q * 16 + e, sl] = rows[q * 16 + e, sl] * sc_
            return cz
        lax.fori_loop(0, _K2 // 16, body, 0, unroll=2)

    def clrcp(i):
        return pltpu.make_async_copy(zbuf, acc.at[pl.ds(my0 + i * 16, 16)], csem)

    def per_t(t, carry):
        def clr(i, cy):
            clrcp(i).start()
            return cy
        lax.fori_loop(0, _RPW // 16, clr, 0)

        def clw(i, cy):
            clrcp(i).wait()
            return cy
        lax.fori_loop(0, _RPW // 16, clw, 0)
        plsc.subcore_barrier()

        gath(0, rows0, gsem0).start()
        dfetch(0, dstc0, dsem0).start()
        gath(1, rows1, gsem1).start()
        dfetch(1, dstc1, dsem1).start()
        _NT = _CH2 // 3

        def triple(i, cy):
            a = 3 * i

            gath(a, rows0, gsem0).wait()
            scale(a, rows0)
            dfetch(a, dstc0, dsem0).wait()
            pltpu.async_copy(rows0, acc.at[dstc0], ssem0, add=True)

            @pl.when(i > 0)
            def _():
                scat(rows2, dstc2, ssem2).wait()
            gath(a + 2, rows2, gsem2).start()
            dfetch(a + 2, dstc2, dsem2).start()

            gath(a + 1, rows1, gsem1).wait()
            scale(a + 1, rows1)
            dfetch(a + 1, dstc1, dsem1).wait()
            pltpu.async_copy(rows1, acc.at[dstc1], ssem1, add=True)

            @pl.when(i + 1 < _NT)
            def _():
                scat(rows0, dstc0, ssem0).wait()
                gath(a + 3, rows0, gsem0).start()
                dfetch(a + 3, dstc0, dsem0).start()

            gath(a + 2, rows2, gsem2).wait()
            scale(a + 2, rows2)
            dfetch(a + 2, dstc2, dsem2).wait()
            pltpu.async_copy(rows2, acc.at[dstc2], ssem2, add=True)

            @pl.when(i + 1 < _NT)
            def _():
                scat(rows1, dstc1, ssem1).wait()
                gath(a + 4, rows1, gsem1).start()
                dfetch(a + 4, dstc1, dsem1).start()
            return cy
        lax.fori_loop(0, _NT, triple, 0)
        scat(rows0, dstc0, ssem0).wait()
        scat(rows1, dstc1, ssem1).wait()
        scat(rows2, dstc2, ssem2).wait()
        plsc.subcore_barrier()
        pltpu.sync_copy(acc.at[pl.ds(my0, _RPW)],
                        out_hbm.at[c, t, pl.ds(my0, _RPW)])

        def bump(i, cy):
            r = i // 8
            sl = pl.ds((i % 8) * 16, 16)
            idx2d[r, sl] = idx2d[r, sl] + _NP
            return cy
        lax.fori_loop(0, _CH * (_K // 16), bump, 0)
        return carry
    lax.fori_loop(0, _T, per_t, 0)


def kernel(x, edge_index, edge_attr, l0_tw, l0_tb, l0_g, l0_b, l0_gw, l0_gb,
           l1_tw, l1_tb, l1_g, l1_b, l1_gw, l1_gb, out_w, out_b):
    f32 = jnp.float32
    ew = jnp.squeeze(edge_attr, axis=-1)
    ew_soft = _softmax(ew.reshape(_E // 128, 128)).reshape(_E)
    src = edge_index[0]
    dst = edge_index[1]
    loopi = jnp.arange(_N, dtype=jnp.int32)
    npad = _E2P - _E2
    src2 = jnp.concatenate([src, loopi, jnp.zeros((npad,), jnp.int32)])
    dst2 = jnp.concatenate([dst, loopi, jnp.zeros((npad,), jnp.int32)])
    ew2 = jnp.concatenate([ew_soft, jnp.ones((_N,), f32), jnp.zeros((npad,), f32)])
    srcp = src2.reshape(_NW, _EPW)
    dstp = dst2.reshape(_NW, _EPW)
    srcp3 = src2.reshape(_NW, _CH, _K)
    dstp3 = dst2.reshape(_NW, _CH2, _K2)
    ewp = ew2.reshape(_NW, _EPW)

    deg_part = _deg_kernel(dstp, ewp)
    dinv = _dinv(deg_part).reshape(_NP)
    normp = _norm_kernel(srcp, dstp, ewp, dinv)

    xpad = jnp.pad(x, ((0, 0), (0, _NP - _N), (0, 0)))
    w00 = l0_tw[:, :, 0].T
    w01 = l0_tw[:, :, 1].T
    w02 = l0_tw[:, :, 2].T
    w10 = l1_tw[:, :, 0].T
    w11 = l1_tw[:, :, 1].T
    w12 = l1_tw[:, :, 2].T
    r1 = lambda a: a.reshape(1, _H)

    xw0 = _dense0(xpad, w00, w01, w02, r1(l0_tb), r1(l0_g), r1(l0_b), l0_gw)
    agg0 = _mp_kernel(xw0.reshape(_T * _NP, _H), srcp3, dstp3, normp)
    xw1 = _dense1(agg0, r1(l0_gb), w10, w11, w12, r1(l1_tb), r1(l1_g), r1(l1_b), l1_gw)
    agg1 = _mp_kernel(xw1.reshape(_T * _NP, _H), srcp3, dstp3, normp)
    outp = _proj_out(agg1, r1(l1_gb), out_w, r1(out_b))
    return outp[:_N]


# speedup vs baseline: 1.8777x; 1.0006x over previous
import functools

import jax
import jax.numpy as jnp
from jax import lax
from jax.experimental import pallas as pl
from jax.experimental.pallas import tpu as pltpu
from jax.experimental.pallas import tpu_sc as plsc

_N = 10000
_NP = 10240
_E = 320000
_E2 = _E + _N
_T = 12
_H = 128
_NW = 32
_K = 128
_CH = -(-_E2 // (_NW * _K))
_K2 = 64
_CH2 = 2 * _CH
_EPW = _CH * _K
_E2P = _NW * _EPW
_RPW = _NP // 16

_mesh = plsc.VectorSubcoreMesh(core_axis_name="c", subcore_axis_name="s")


def _softmax_body(ew_ref, out_ref):
    v = ew_ref[...]
    m = jnp.max(v)
    e = jnp.exp(v - m)
    out_ref[...] = e / jnp.sum(e)


def _softmax(ew2d):
    return pl.pallas_call(
        _softmax_body,
        out_shape=jax.ShapeDtypeStruct(ew2d.shape, jnp.float32),
    )(ew2d)


def _dinv_body(dp_ref, out_ref):
    d = jnp.sum(dp_ref[...], axis=0)
    out_ref[...] = jnp.where(d > 0, lax.rsqrt(d), 0.0).reshape(_NP // 128, 128)


def _dinv(deg_part):
    return pl.pallas_call(
        _dinv_body,
        out_shape=jax.ShapeDtypeStruct((_NP // 128, 128), jnp.float32),
    )(deg_part)


def _dense_core(xb, w0, w1, w2, tb, g, b, gw):
    tt, nb, _ = xb.shape
    y = xb.reshape(tt * nb, -1)
    a = jnp.dot(y, w0, preferred_element_type=jnp.float32).reshape(tt, nb, -1)
    bb = jnp.dot(y, w1, preferred_element_type=jnp.float32).reshape(tt, nb, -1)
    cc = jnp.dot(y, w2, preferred_element_type=jnp.float32).reshape(tt, nb, -1)
    z = jnp.zeros((1, nb, a.shape[-1]), jnp.float32)
    xc = bb + jnp.concatenate([z, a[:-1]], 0) + jnp.concatenate([cc[1:], z], 0)
    xc = jnp.clip(xc + tb.reshape(1, 1, -1), -10.0, 10.0)
    m = jnp.mean(xc, axis=0, keepdims=True)
    v = jnp.mean((xc - m) ** 2, axis=0, keepdims=True)
    xc = (xc - m) * lax.rsqrt(v + 1e-5) * g.reshape(1, 1, -1) + b.reshape(1, 1, -1)
    xc = jnp.maximum(xc, 0.0)
    xw = jnp.dot(xc.reshape(tt * nb, -1), gw, preferred_element_type=jnp.float32)
    return xw.reshape(tt, nb, -1)


def _dense0_body(x_ref, w0_ref, w1_ref, w2_ref, tb_ref, g_ref, b_ref, gw_ref, out_ref):
    out_ref[...] = _dense_core(x_ref[...], w0_ref[...], w1_ref[...], w2_ref[...],
                               tb_ref[...], g_ref[...], b_ref[...], gw_ref[...])


def _merge_agg(agg, gb):
    h = agg[0] + agg[1] + gb.reshape(1, 1, -1)
    return jnp.clip(jnp.maximum(h, 0.0), -10.0, 10.0)


def _dense1_body(agg_ref, gb_ref, w0_ref, w1_ref, w2_ref, tb_ref, g_ref, b_ref,
                 gw_ref, out_ref):
    h = _merge_agg(agg_ref[...], gb_ref[...])
    out_ref[...] = _dense_core(h, w0_ref[...], w1_ref[...], w2_ref[...],
                               tb_ref[...], g_ref[...], b_ref[...], gw_ref[...])


def _out_body(agg_ref, gb_ref, ow_ref, ob_ref, out_ref):
    h = _merge_agg(agg_ref[...], gb_ref[...])
    hbar = jnp.mean(h, axis=0)
    out_ref[...] = jnp.dot(hbar, ow_ref[...], preferred_element_type=jnp.float32) \
        + ob_ref[...].reshape(1, -1)


_NB = 512
_GRID = _NP // _NB


def _wspec(shape):
    nd = len(shape)
    return pl.BlockSpec(shape, lambda i: (0,) * nd)


def _dense0(xpad, w0, w1, w2, tb, g, b, gw):
    return pl.pallas_call(
        _dense0_body,
        grid=(_GRID,),
        in_specs=[
            pl.BlockSpec((_T, _NB, _H), lambda i: (0, i, 0)),
            _wspec((_H, _H)), _wspec((_H, _H)), _wspec((_H, _H)),
            _wspec((1, _H)), _wspec((1, _H)), _wspec((1, _H)), _wspec((_H, _H)),
        ],
        out_specs=pl.BlockSpec((_T, _NB, _H), lambda i: (0, i, 0)),
        out_shape=jax.ShapeDtypeStruct((_T, _NP, _H), jnp.float32),
    )(xpad, w0, w1, w2, tb, g, b, gw)


def _dense1(agg, gb, w0, w1, w2, tb, g, b, gw):
    return pl.pallas_call(
        _dense1_body,
        grid=(_GRID,),
        in_specs=[
            pl.BlockSpec((2, _T, _NB, _H), lambda i: (0, 0, i, 0)),
            _wspec((1, _H)),
            _wspec((_H, _H)), _wspec((_H, _H)), _wspec((_H, _H)),
            _wspec((1, _H)), _wspec((1, _H)), _wspec((1, _H)), _wspec((_H, _H)),
        ],
        out_specs=pl.BlockSpec((_T, _NB, _H), lambda i: (0, i, 0)),
        out_shape=jax.ShapeDtypeStruct((_T, _NP, _H), jnp.float32),
    )(agg, gb, w0, w1, w2, tb, g, b, gw)


def _proj_out(agg, gb, ow, ob):
    return pl.pallas_call(
        _out_body,
        grid=(_GRID,),
        in_specs=[
            pl.BlockSpec((2, _T, _NB, _H), lambda i: (0, 0, i, 0)),
            _wspec((1, _H)), _wspec((_H, _H)), _wspec((1, _H)),
        ],
        out_specs=pl.BlockSpec((_NB, _H), lambda i: (i, 0)),
        out_shape=jax.ShapeDtypeStruct((_NP, _H), jnp.float32),
    )(agg, gb, ow, ob)


@functools.partial(
    pl.kernel,
    out_type=jax.ShapeDtypeStruct((_NW, _NP), jnp.float32),
    mesh=_mesh,
    compiler_params=pltpu.CompilerParams(needs_layout_passes=False),
    scratch_types=[
        pltpu.VMEM((_EPW,), jnp.int32),
        pltpu.VMEM((_EPW,), jnp.float32),
        pltpu.VMEM((_NP,), jnp.float32),
    ],
)
def _deg_kernel(dst_hbm, ew_hbm, out_hbm, dstv, ewv, degv):
    c = lax.axis_index("c")
    s = lax.axis_index("s")
    w = c * 16 + s
    pltpu.sync_copy(dst_hbm.at[w], dstv)
    pltpu.sync_copy(ew_hbm.at[w], ewv)

    def zero(i, carry):
        degv[pl.ds(i * 16, 16)] = jnp.zeros((16,), jnp.float32)
        return carry
    lax.fori_loop(0, _NP // 16, zero, 0)

    def step(i, carry):
        idx = dstv[pl.ds(i * 16, 16)]
        wv = ewv[pl.ds(i * 16, 16)]
        plsc.addupdate_scatter(degv, [idx], wv)
        return carry
    lax.fori_loop(0, _EPW // 16, step, 0)
    pltpu.sync_copy(degv, out_hbm.at[w])


@functools.partial(
    pl.kernel,
    out_type=jax.ShapeDtypeStruct((_NW, _EPW), jnp.float32),
    mesh=_mesh,
    compiler_params=pltpu.CompilerParams(needs_layout_passes=False),
    scratch_types=[
        pltpu.VMEM((_EPW,), jnp.int32),
        pltpu.VMEM((_EPW,), jnp.int32),
        pltpu.VMEM((_EPW,), jnp.float32),
        pltpu.VMEM((_NP,), jnp.float32),
        pltpu.VMEM((_EPW,), jnp.float32),
    ],
)
def _norm_kernel(src_hbm, dst_hbm, ew_hbm, dinv_hbm, out_hbm,
                 srcv, dstv, ewv, dinvv, outv):
    c = lax.axis_index("c")
    s = lax.axis_index("s")
    w = c * 16 + s
    pltpu.sync_copy(src_hbm.at[w], srcv)
    pltpu.sync_copy(dst_hbm.at[w], dstv)
    pltpu.sync_copy(ew_hbm.at[w], ewv)
    pltpu.sync_copy(dinv_hbm, dinvv)

    def step(i, carry):
        sl = pl.ds(i * 16, 16)
        a = plsc.load_gather(dinvv, [srcv[sl]])
        bq = plsc.load_gather(dinvv, [dstv[sl]])
        outv[sl] = a * ewv[sl] * bq
        return carry
    lax.fori_loop(0, _EPW // 16, step, 0)
    pltpu.sync_copy(outv, out_hbm.at[w])


@functools.partial(
    pl.kernel,
    out_type=jax.ShapeDtypeStruct((2, _T, _NP, _H), jnp.float32),
    mesh=_mesh,
    compiler_params=pltpu.CompilerParams(needs_layout_passes=False),
    scratch_types=[
        pltpu.VMEM((_CH, _K), jnp.int32),
        pltpu.VMEM((_EPW,), jnp.float32),
        pltpu.VMEM((_K2,), jnp.int32),
        pltpu.VMEM((_K2,), jnp.int32),
        pltpu.VMEM((_K2,), jnp.int32),
        pltpu.VMEM((_K2, _H), jnp.float32),
        pltpu.VMEM((_K2, _H), jnp.float32),
        pltpu.VMEM((_K2, _H), jnp.float32),
        pltpu.VMEM((16, _H), jnp.float32),
        pltpu.VMEM_SHARED((_NP, _H), jnp.float32),
        pltpu.SemaphoreType.DMA,
        pltpu.SemaphoreType.DMA,
        pltpu.SemaphoreType.DMA,
        pltpu.SemaphoreType.DMA,
        pltpu.SemaphoreType.DMA,
        pltpu.SemaphoreType.DMA,
        pltpu.SemaphoreType.DMA,
        pltpu.SemaphoreType.DMA,
        pltpu.SemaphoreType.DMA,
        pltpu.SemaphoreType.DMA,
    ],
)
def _mp_kernel(xw_hbm, src_hbm, dst_hbm, norm_hbm, out_hbm,
               idx2d, normv, dstc0, dstc1, dstc2, rows0, rows1, rows2,
               zbuf, acc, gsem0, gsem1, gsem2, ssem0, ssem1, ssem2,
               dsem0, dsem1, dsem2, csem):
    c = lax.axis_index("c")
    s = lax.axis_index("s")
    w = c * 16 + s
    pltpu.sync_copy(src_hbm.at[w], idx2d)
    pltpu.sync_copy(norm_hbm.at[w], normv)

    def zb(i, carry):
        zbuf[i // 8, pl.ds((i % 8) * 16, 16)] = jnp.zeros((16,), jnp.float32)
        return carry
    lax.fori_loop(0, 16 * 8, zb, 0)
    my0 = s * _RPW

    def _gath2(j, rows, gsem):
        row = j // 2
        base = (j % 2) * _K2
        ia = idx2d.at[row, pl.ds(base, _K2 // 2)]
        ib = idx2d.at[row, pl.ds(base + _K2 // 2, _K2 // 2)]
        ca = pltpu.make_async_copy(xw_hbm.at[ia], rows.at[pl.ds(0, _K2 // 2)], gsem)
        cb = pltpu.make_async_copy(xw_hbm.at[ib], rows.at[pl.ds(_K2 // 2, _K2 // 2)], gsem)
        return ca, cb

    class _GPair:
        def __init__(self, a, b):
            self._a, self._b = a, b

        def start(self):
            self._a.start()
            self._b.start()

        def wait(self):
            self._a.wait()
            self._b.wait()

    def gath(j, rows, gsem):
        return _GPair(*_gath2(j, rows, gsem))

    def dfetch(j, dstc, dsem):
        return pltpu.make_async_copy(dst_hbm.at[w, j], dstc, dsem)

    def scat(rows, dstc, ssem):
        return pltpu.make_async_copy(rows, acc.at[dstc], ssem)

    def scale(cidx, rows):
        def body(q, cz):
            nv = normv[pl.ds(cidx * _K2 + q * 16, 16)]
            for e in range(16):
                sc_ = nv[e]
                for k in range(_H // 16):
                    sl = pl.ds(k * 16, 16)
                    rows[q * 16 + e, sl] = rows[q * 16 + e, sl] * sc_
            return cz
        lax.fori_loop(0, _K2 // 16, body, 0, unroll=2)

    def clrcp(i):
        return pltpu.make_async_copy(zbuf, acc.at[pl.ds(my0 + i * 16, 16)], csem)

    def per_t(t, carry):
        def clr(i, cy):
            clrcp(i).start()
            return cy
        lax.fori_loop(0, _RPW // 16, clr, 0)

        def clw(i, cy):
            clrcp(i).wait()
            return cy
        lax.fori_loop(0, _RPW // 16, clw, 0)
        plsc.subcore_barrier()

        gath(0, rows0, gsem0).start()
        dfetch(0, dstc0, dsem0).start()
        gath(1, rows1, gsem1).start()
        dfetch(1, dstc1, dsem1).start()
        _NT = _CH2 // 3

        def triple(i, cy):
            a = 3 * i

            gath(a, rows0, gsem0).wait()
            scale(a, rows0)
            dfetch(a, dstc0, dsem0).wait()
            pltpu.async_copy(rows0, acc.at[dstc0], ssem0, add=True)

            @pl.when(i > 0)
            def _():
                scat(rows2, dstc2, ssem2).wait()
            gath(a + 2, rows2, gsem2).start()
            dfetch(a + 2, dstc2, dsem2).start()

            gath(a + 1, rows1, gsem1).wait()
            scale(a + 1, rows1)
            dfetch(a + 1, dstc1, dsem1).wait()
            pltpu.async_copy(rows1, acc.at[dstc1], ssem1, add=True)

            @pl.when(i + 1 < _NT)
            def _():
                scat(rows0, dstc0, ssem0).wait()
                gath(a + 3, rows0, gsem0).start()
                dfetch(a + 3, dstc0, dsem0).start()

            gath(a + 2, rows2, gsem2).wait()
            scale(a + 2, rows2)
            dfetch(a + 2, dstc2, dsem2).wait()
            pltpu.async_copy(rows2, acc.at[dstc2], ssem2, add=True)

            @pl.when(i + 1 < _NT)
            def _():
                scat(rows1, dstc1, ssem1).wait()
                gath(a + 4, rows1, gsem1).start()
                dfetch(a + 4, dstc1, dsem1).start()
            return cy
        lax.fori_loop(0, _NT, triple, 0)
        scat(rows0, dstc0, ssem0).wait()
        scat(rows1, dstc1, ssem1).wait()
        scat(rows2, dstc2, ssem2).wait()
        plsc.subcore_barrier()
        pltpu.sync_copy(acc.at[pl.ds(my0, _RPW)],
                        out_hbm.at[c, t, pl.ds(my0, _RPW)])

        def bump(i, cy):
            r = i // 8
            sl = pl.ds((i % 8) * 16, 16)
            idx2d[r, sl] = idx2d[r, sl] + _NP
            return cy
        lax.fori_loop(0, _CH * (_K // 16), bump, 0)
        return carry
    lax.fori_loop(0, _T, per_t, 0)


def kernel(x, edge_index, edge_attr, l0_tw, l0_tb, l0_g, l0_b, l0_gw, l0_gb,
           l1_tw, l1_tb, l1_g, l1_b, l1_gw, l1_gb, out_w, out_b):
    f32 = jnp.float32
    ew = jnp.squeeze(edge_attr, axis=-1)
    ew_soft = _softmax(ew.reshape(_E // 128, 128)).reshape(_E)
    src = edge_index[0]
    dst = edge_index[1]
    loopi = jnp.arange(_N, dtype=jnp.int32)
    npad = _E2P - _E2
    src2 = jnp.concatenate([src, loopi, jnp.zeros((npad,), jnp.int32)])
    dst2 = jnp.concatenate([dst, loopi, jnp.zeros((npad,), jnp.int32)])
    ew2 = jnp.concatenate([ew_soft, jnp.ones((_N,), f32), jnp.zeros((npad,), f32)])
    srcp = src2.reshape(_NW, _EPW)
    dstp = dst2.reshape(_NW, _EPW)
    srcp3 = src2.reshape(_NW, _CH, _K)
    dstp3 = dst2.reshape(_NW, _CH2, _K2)
    ewp = ew2.reshape(_NW, _EPW)

    deg_part = _deg_kernel(dstp, ewp)
    dinv = _dinv(deg_part).reshape(_NP)
    normp = _norm_kernel(srcp, dstp, ewp, dinv)

    xpad = jnp.pad(x, ((0, 0), (0, _NP - _N), (0, 0)))
    w00 = l0_tw[:, :, 0].T
    w01 = l0_tw[:, :, 1].T
    w02 = l0_tw[:, :, 2].T
    w10 = l1_tw[:, :, 0].T
    w11 = l1_tw[:, :, 1].T
    w12 = l1_tw[:, :, 2].T
    r1 = lambda a: a.reshape(1, _H)

    xw0 = _dense0(xpad, w00, w01, w02, r1(l0_tb), r1(l0_g), r1(l0_b), l0_gw)
    agg0 = _mp_kernel(xw0.reshape(_T * _NP, _H), srcp3, dstp3, normp)
    xw1 = _dense1(agg0, r1(l0_gb), w10, w11, w12, r1(l1_tb), r1(l1_g), r1(l1_b), l1_gw)
    agg1 = _mp_kernel(xw1.reshape(_T * _NP, _H), srcp3, dstp3, normp)
    outp = _proj_out(agg1, r1(l1_gb), out_w, r1(out_b))
    return outp[:_N]
